# baseline (speedup 1.0000x reference)
import functools

import jax
import jax.numpy as jnp
from jax import lax
from jax.experimental import pallas as pl
from jax.experimental.pallas import tpu as pltpu
from jax.experimental.pallas import tpu_sc as plsc

B = 1024
DIM = 512
MAXN = 64
NC, NS, L = 2, 16, 16
NW = NC * NS
BPW = B // NW
NG = MAXN // L
DC = DIM // L
NA = 8


def _make(cnt_r, interpret=False):
  mesh = plsc.VectorSubcoreMesh(core_axis_name="c", subcore_axis_name="s",
                                num_cores=NC, num_subcores=NS)

  @functools.partial(
      pl.kernel,
      out_type=jax.ShapeDtypeStruct((B, DIM), jnp.float32),
      mesh=mesh,
      scratch_types=[
          pltpu.VMEM((BPW,), jnp.int32),
          pltpu.VMEM((BPW * MAXN * 2,), jnp.int32),
          pltpu.VMEM((BPW * MAXN,), jnp.int32),
          pltpu.VMEM((BPW * MAXN * 2,), jnp.int32),
          pltpu.VMEM((BPW * MAXN,), jnp.float32),
          pltpu.VMEM((cnt_r,), jnp.float32),
          pltpu.VMEM((BPW, MAXN), jnp.float32),
          pltpu.VMEM((BPW, MAXN), jnp.int32),
          pltpu.VMEM((MAXN, DIM), jnp.float32),
          pltpu.VMEM((MAXN, DIM), jnp.float32),
          pltpu.VMEM((BPW, DIM), jnp.float32),
          pltpu.SemaphoreType.DMA,
          pltpu.SemaphoreType.DMA,
      ],
      compiler_params=pltpu.CompilerParams(needs_layout_passes=False),
      interpret=interpret,
  )
  def k(hp_hbm, tg_hbm, tw_hbm, corr_hbm, emb_hbm, out_hbm,
        hp_v, gidx_v, widx_v, nei_v, tw_v, corr_v, w_v, eid_v,
        rows0, rows1, out_v, sem0, sem1):
    wid = lax.axis_index("s") * NC + lax.axis_index("c")
    base = wid * BPW
    pltpu.sync_copy(hp_hbm.at[pl.ds(base, BPW)], hp_v)

    lane = lax.iota(jnp.int32, L)
    zero16 = jnp.zeros((L,), jnp.int32)
    one16 = jnp.full((L,), 1, jnp.int32)

    def ibody(b, _):
      hpb = plsc.load_gather(hp_v, [zero16 + b])
      gbase = hpb * (MAXN * 2)
      wbase = hpb * MAXN
      for ck in range(MAXN * 2 // L):
        gidx_v[pl.ds(b * (MAXN * 2) + ck * L, L)] = gbase + ck * L + lane
      for ck in range(MAXN // L):
        widx_v[pl.ds(b * MAXN + ck * L, L)] = wbase + ck * L + lane
      return 0
    lax.fori_loop(0, BPW, ibody, 0)

    pltpu.async_copy(tg_hbm.at[gidx_v], nei_v, sem0)
    pltpu.async_copy(tw_hbm.at[widx_v], tw_v, sem1)
    pltpu.sync_copy(corr_hbm, corr_v)
    pltpu.make_async_copy(tg_hbm.at[gidx_v], nei_v, sem0).wait()
    pltpu.make_async_copy(tw_hbm.at[widx_v], tw_v, sem1).wait()

    def wbody(b, _):
      for g in range(NG):
        fidx = b * (MAXN * 2) + g * (L * 2) + lane * 2
        rid = plsc.load_gather(nei_v, [fidx])
        eid = plsc.load_gather(nei_v, [fidx + one16])
        num = plsc.load_gather(corr_v, [rid])
        den = tw_v[pl.ds(b * MAXN + g * L, L)]
        w_v[b, pl.ds(g * L, L)] = num / den
        eid_v[b, pl.ds(g * L, L)] = eid
      return 0
    lax.fori_loop(0, BPW, wbody, 0)

    pltpu.async_copy(emb_hbm.at[eid_v.at[0]], rows0, sem0)
    pltpu.async_copy(emb_hbm.at[eid_v.at[1]], rows1, sem1)

    def compute(b, rows_p):
      wg = [w_v[b, pl.ds(g * L, L)] for g in range(NG)]
      def dbody(dc, _):
        accs = [jnp.zeros((L,), jnp.float32) for _ in range(NA)]
        for n in range(MAXN):
          accs[n % NA] = (accs[n % NA]
                          + wg[n // L][n % L] * rows_p[n, pl.ds(dc * L, L)])
        while len(accs) > 1:
          accs = [a + c for a, c in zip(accs[0::2], accs[1::2])]
        out_v[b, pl.ds(dc * L, L)] = accs[0]
        return 0
      lax.fori_loop(0, DC, dbody, 0)

    def mbody(bb, _):
      for p, rows_p, sem_p in ((0, rows0, sem0), (1, rows1, sem1)):
        b = bb * 2 + p
        pltpu.make_async_copy(emb_hbm.at[eid_v.at[b]], rows_p, sem_p).wait()
        compute(b, rows_p)
        @pl.when(b + 2 < BPW)
        def _():
          pltpu.async_copy(emb_hbm.at[eid_v.at[b + 2]], rows_p, sem_p)
      return 0
    lax.fori_loop(0, BPW // 2, mbody, 0)

    pltpu.sync_copy(out_v, out_hbm.at[pl.ds(base, BPW)])

  return k


@jax.jit
def kernel(hp, rp, tp, hn, rn, tn, e_emb, train_w, corr, train_g):
  del rp, tp, hn, rn, tn
  k = _make(corr.shape[0])
  return k(hp.astype(jnp.int32), train_g.astype(jnp.int32).ravel(),
           train_w.ravel(), corr, e_emb)


# speedup vs baseline: 26.7032x; 26.7032x over previous
import functools

import jax
import jax.numpy as jnp
from jax import lax
from jax.experimental import pallas as pl
from jax.experimental.pallas import tpu as pltpu
from jax.experimental.pallas import tpu_sc as plsc

B = 1024
DIM = 512
MAXN = 64
NC, NS, L = 2, 16, 16
NW = NC * NS
BPW = B // NW
NG = MAXN // L
DC = DIM // L
NA = 8


def _take_rows(table, idx):
  dnums = lax.GatherDimensionNumbers(
      offset_dims=tuple(range(1, table.ndim)),
      collapsed_slice_dims=(0,),
      start_index_map=(0,))
  return lax.gather(table, idx[:, None], dnums,
                    (1,) + table.shape[1:],
                    mode=lax.GatherScatterMode.PROMISE_IN_BOUNDS)


def _make(cnt_r, interpret=False):
  mesh = plsc.VectorSubcoreMesh(core_axis_name="c", subcore_axis_name="s",
                                num_cores=NC, num_subcores=NS)

  @functools.partial(
      pl.kernel,
      out_type=jax.ShapeDtypeStruct((B, DIM), jnp.float32),
      mesh=mesh,
      scratch_types=[
          pltpu.VMEM((BPW, MAXN * 2), jnp.int32),
          pltpu.VMEM((BPW, MAXN), jnp.float32),
          pltpu.VMEM((cnt_r,), jnp.float32),
          pltpu.VMEM((BPW, MAXN), jnp.float32),
          pltpu.VMEM((BPW, MAXN), jnp.int32),
          pltpu.VMEM((MAXN, DIM), jnp.float32),
          pltpu.VMEM((MAXN, DIM), jnp.float32),
          pltpu.VMEM((BPW, DIM), jnp.float32),
          pltpu.SemaphoreType.DMA,
          pltpu.SemaphoreType.DMA,
      ],
      compiler_params=pltpu.CompilerParams(needs_layout_passes=False),
      interpret=interpret,
  )
  def k(nei_hbm, den_hbm, corr_hbm, emb_hbm, out_hbm,
        nei_v, tw_v, corr_v, w_v, eid_v, rows0, rows1, out_v,
        sem0, sem1):
    wid = lax.axis_index("s") * NC + lax.axis_index("c")
    base = wid * BPW
    pltpu.sync_copy(nei_hbm.at[pl.ds(base, BPW)], nei_v)
    pltpu.sync_copy(den_hbm.at[pl.ds(base, BPW)], tw_v)
    pltpu.sync_copy(corr_hbm, corr_v)

    lane = lax.iota(jnp.int32, L)
    zero16 = jnp.zeros((L,), jnp.int32)
    one16 = jnp.full((L,), 1, jnp.int32)

    def weights(b):
      b16 = zero16 + b
      for g in range(NG):
        colr = g * (L * 2) + lane * 2
        rid = plsc.load_gather(nei_v, [b16, colr])
        eid = plsc.load_gather(nei_v, [b16, colr + one16])
        num = plsc.load_gather(corr_v, [rid])
        den = tw_v[b, pl.ds(g * L, L)]
        w_v[b, pl.ds(g * L, L)] = num / den
        eid_v[b, pl.ds(g * L, L)] = eid

    weights(0)
    weights(1)
    pltpu.async_copy(emb_hbm.at[eid_v.at[0]], rows0, sem0)
    pltpu.async_copy(emb_hbm.at[eid_v.at[1]], rows1, sem1)

    def compute(b, rows_p):
      wg = [w_v[b, pl.ds(g * L, L)] for g in range(NG)]
      def dbody(dc, _):
        accs = [jnp.zeros((L,), jnp.float32) for _ in range(NA)]
        for n in range(MAXN):
          accs[n % NA] = (accs[n % NA]
                          + wg[n // L][n % L] * rows_p[n, pl.ds(dc * L, L)])
        while len(accs) > 1:
          accs = [a + c for a, c in zip(accs[0::2], accs[1::2])]
        out_v[b, pl.ds(dc * L, L)] = accs[0]
        return 0
      lax.fori_loop(0, DC, dbody, 0)

    def mbody(bb, _):
      for p, rows_p, sem_p in ((0, rows0, sem0), (1, rows1, sem1)):
        b = bb * 2 + p
        @pl.when(b + 2 < BPW)
        def _():
          weights(b + 2)
        pltpu.make_async_copy(emb_hbm.at[eid_v.at[b]], rows_p, sem_p).wait()
        compute(b, rows_p)
        @pl.when(b + 2 < BPW)
        def _():
          pltpu.async_copy(emb_hbm.at[eid_v.at[b + 2]], rows_p, sem_p)
      return 0
    lax.fori_loop(0, BPW // 2, mbody, 0)

    pltpu.sync_copy(out_v, out_hbm.at[pl.ds(base, BPW)])

  return k


@jax.jit
def kernel(hp, rp, tp, hn, rn, tn, e_emb, train_w, corr, train_g):
  del rp, tp, hn, rn, tn
  k = _make(corr.shape[0])
  nei = _take_rows(train_g.astype(jnp.int32), hp)
  den = _take_rows(train_w, hp)
  return k(nei.reshape(B, MAXN * 2), den, corr, e_emb)
